# baseline (speedup 1.0000x reference)
import jax
import jax.numpy as jnp
from jax.experimental import pallas as pl
from jax.experimental.pallas import tpu as pltpu

_NUM_EMB = 512
_EMB_DIM = 64
_BETA = 0.25


def _patch_mm_body(x_ref, w_ref, b_ref, o_ref):
    _, H, _, kdim = x_ref.shape
    co = o_ref.shape[3]
    acc = jnp.dot(x_ref[0].reshape(H * H, kdim), w_ref[...],
                  preferred_element_type=jnp.float32)
    o_ref[0] = jnp.maximum(acc + b_ref[0], 0.0).reshape(H, H, co)


def _mid_body(h0_ref, wm3_ref, b3_ref, r1a_ref, r1b_ref, r2a_ref, r2b_ref,
              pre_ref, preb_ref, cb_ref, cbt_ref, d1_ref, db1_ref,
              dr1a_ref, dr1b_ref, dr2a_ref, dr2b_ref,
              out_ref, loss_ref, sp, sc):
    i = pl.program_id(0)

    @pl.when(i == 0)
    def _init():
        loss_ref[...] = jnp.zeros((1, 1), jnp.float32)

    sp[...] = jnp.zeros((58, 58, 128), jnp.float32)

    def conv3(w_ref, cin, cout):
        for dy in range(3):
            for dx in range(3):
                k0 = (dy * 3 + dx) * cin
                sc[:, :, k0:k0 + cin] = sp[dy:dy + 56, dx:dx + 56, 0:cin]
        patches = sc[:, :, 0:9 * cin].reshape(3136, 9 * cin)
        return jnp.dot(patches, w_ref[...], preferred_element_type=jnp.float32)

    def res_block(h, ra_ref, rb_ref):
        t = jnp.maximum(h, 0.0)
        sp[1:57, 1:57, :] = t.reshape(56, 56, 128)
        u = jnp.maximum(conv3(ra_ref, 128, 32), 0.0)
        return h + jnp.dot(u, rb_ref[...], preferred_element_type=jnp.float32)

    sp[1:57, 1:57, :] = h0_ref[0]
    h = conv3(wm3_ref, 128, 128) + b3_ref[0]
    h = res_block(h, r1a_ref, r1b_ref)
    h = res_block(h, r2a_ref, r2b_ref)
    h = jnp.maximum(h, 0.0)

    z = jnp.dot(h, pre_ref[...], preferred_element_type=jnp.float32) + preb_ref[0]

    cb = cb_ref[...]
    cb2 = jnp.sum(cb * cb, axis=1)
    sse = jnp.zeros((1, 1), jnp.float32)
    for c in range(4):
        zc = z[c * 784:(c + 1) * 784, :]
        z2 = jnp.sum(zc * zc, axis=1, keepdims=True)
        d = (z2 + cb2[None, :]) - 2.0 * jnp.dot(zc, cbt_ref[...],
                                                preferred_element_type=jnp.float32)
        m = jnp.min(d, axis=1, keepdims=True)
        iota = jax.lax.broadcasted_iota(jnp.int32, (784, _NUM_EMB), 1)
        masked = jnp.where(d <= m, iota, _NUM_EMB)
        idx = jnp.min(masked, axis=1, keepdims=True)
        onehot = jnp.where(iota == idx, 1.0, 0.0)
        qc = jnp.dot(onehot, cb, preferred_element_type=jnp.float32)
        diff = qc - zc
        sse += jnp.sum(diff * diff).reshape(1, 1)
        sp[1 + c * 14:1 + (c + 1) * 14, 1:57, 0:64] = qc.reshape(14, 56, 64)
    loss_ref[...] += sse

    h = conv3(d1_ref, 64, 128) + db1_ref[0]
    h = res_block(h, dr1a_ref, dr1b_ref)
    h = res_block(h, dr2a_ref, dr2b_ref)
    h = jnp.maximum(h, 0.0)
    out_ref[0] = h.reshape(56, 56, 128)


def _tconv_body(cout, relu):
    def body(x_ref, w_ref, b_ref, o_ref):
        _, H, _, _ = o_ref.shape
        cin = x_ref.shape[3]
        for r in range(2):
            for s in range(2):
                acc = jnp.zeros((H * H, cout), jnp.float32)
                for a in range(2):
                    for b in range(2):
                        xs = x_ref[0, r + a:r + a + H, s + b:s + b + H, :]
                        acc += jnp.dot(xs.reshape(H * H, cin),
                                       w_ref[r * 2 + s, a * 2 + b],
                                       preferred_element_type=jnp.float32)
                acc = acc + b_ref[0]
                if relu:
                    acc = jnp.maximum(acc, 0.0)
                o_ref[0, :, :, (r * 2 + s) * cout:(r * 2 + s + 1) * cout] = (
                    acc.reshape(H, H, cout))
    return body


def _tconv4x4_body(x_ref, w_ref, b_ref, o_ref):
    cin = x_ref.shape[3]
    _, H, _, cq = o_ref.shape
    acc = jnp.zeros((H * H, cq), jnp.float32)
    for dy in range(3):
        for dx in range(3):
            xs = x_ref[0, dy:dy + H, dx:dx + H, :]
            acc += jnp.dot(xs.reshape(H * H, cin), w_ref[dy * 3 + dx],
                           preferred_element_type=jnp.float32)
    o_ref[0] = (acc + b_ref[0]).reshape(H, H, cq)


def _s2d(t):
    b, h, w, c = t.shape
    t = t.reshape(b, h // 2, 2, w // 2, 2, c)
    t = t.transpose(0, 1, 3, 2, 4, 5)
    return t.reshape(b, h // 2, w // 2, 4 * c)


def _d2s(t, c):
    b, h, w, _ = t.shape
    t = t.reshape(b, h, w, 2, 2, c)
    t = t.transpose(0, 1, 3, 2, 4, 5)
    return t.reshape(b, 2 * h, 2 * w, c)


def _lin_w(w):
    o, i, kh, kw = w.shape
    return w.transpose(2, 3, 1, 0).reshape(kh * kw * i, o)


def _tconv_w(w):
    o, i = w.shape[0], w.shape[1]
    wf = w[:, :, ::-1, ::-1]
    t = wf.transpose(2, 3, 1, 0)
    t = t.reshape(2, 2, 2, 2, i, o)
    t = t.transpose(1, 3, 0, 2, 4, 5)
    return t.reshape(4, 4, i, o)


def _img_spec(h, w, c):
    return pl.BlockSpec((1, h, w, c), lambda i: (i, 0, 0, 0))


def _fix(shape):
    nd = len(shape)
    return pl.BlockSpec(shape, lambda i, _nd=nd: (0,) * _nd)


def kernel(x, enc_w1, enc_w2, enc_w3, enc_r1a, enc_r1b, enc_r2a, enc_r2b,
           pre_w, dec_w1, dec_r1a, dec_r1b, dec_r2a, dec_r2b,
           dect_w1, dect_w2, enc_b1, enc_b2, enc_b3, pre_b,
           dec_b1, dect_b1, dect_b2, codebook):
    f32 = jnp.float32
    B = x.shape[0]

    w1m = _lin_w(enc_w1)
    w2m = _lin_w(enc_w2)
    wm3 = _lin_w(enc_w3)
    r1a = _lin_w(enc_r1a)
    r2a = _lin_w(enc_r2a)
    r1b = enc_r1b[:, :, 0, 0].T
    r2b = enc_r2b[:, :, 0, 0].T
    pre = pre_w[:, :, 0, 0].T
    d1m = _lin_w(dec_w1)
    dr1a = _lin_w(dec_r1a)
    dr2a = _lin_w(dec_r2a)
    dr1b = dec_r1b[:, :, 0, 0].T
    dr2b = dec_r2b[:, :, 0, 0].T
    wt1 = _tconv_w(dect_w1)
    wt2 = _tconv_w(dect_w2)
    w5 = jnp.zeros((4, 4, 64, 12), jnp.float32)
    for r in range(2):
        for s in range(2):
            for a in range(2):
                for b in range(2):
                    w5 = w5.at[r + a, s + b, :, (r * 2 + s) * 3:(r * 2 + s) * 3 + 3].set(
                        wt2[r * 2 + s, a * 2 + b])
    w5 = w5[:3, :3].reshape(9, 64, 12)
    cbt = codebook.T

    b1 = enc_b1.reshape(1, 64)
    b2 = enc_b2.reshape(1, 128)
    b3 = enc_b3.reshape(1, 128)
    preb = pre_b.reshape(1, 64)
    db1 = dec_b1.reshape(1, 128)
    tb1 = dect_b1.reshape(1, 64)
    tb2 = jnp.tile(dect_b2, 4).reshape(1, 12)

    def _im2col_s2(t):
        tp = jnp.pad(t, ((0, 0), (1, 1), (1, 1), (0, 0)))
        n = t.shape[1] // 2
        return jnp.concatenate(
            [tp[:, ky:ky + 2 * n - 1:2, kx:kx + 2 * n - 1:2, :]
             for ky in range(4) for kx in range(4)], axis=-1)

    xn = x.transpose(0, 2, 3, 1)
    xs = _im2col_s2(xn)

    h1 = pl.pallas_call(
        _patch_mm_body,
        grid=(B,),
        in_specs=[_img_spec(112, 112, 48), _fix((48, 64)), _fix((1, 64))],
        out_specs=_img_spec(112, 112, 64),
        out_shape=jax.ShapeDtypeStruct((B, 112, 112, 64), f32),
    )(xs, w1m, b1)

    h1s = _im2col_s2(h1)

    h2 = pl.pallas_call(
        _patch_mm_body,
        grid=(B,),
        in_specs=[_img_spec(56, 56, 1024), _fix((1024, 128)), _fix((1, 128))],
        out_specs=_img_spec(56, 56, 128),
        out_shape=jax.ShapeDtypeStruct((B, 56, 56, 128), f32),
    )(h1s, w2m, b2)

    mid, loss_raw = pl.pallas_call(
        _mid_body,
        grid=(B,),
        in_specs=[
            _img_spec(56, 56, 128),
            _fix((1152, 128)), _fix((1, 128)),
            _fix((1152, 32)), _fix((32, 128)),
            _fix((1152, 32)), _fix((32, 128)),
            _fix((128, 64)), _fix((1, 64)),
            _fix((_NUM_EMB, _EMB_DIM)), _fix((_EMB_DIM, _NUM_EMB)),
            _fix((576, 128)), _fix((1, 128)),
            _fix((1152, 32)), _fix((32, 128)),
            _fix((1152, 32)), _fix((32, 128)),
        ],
        out_specs=[_img_spec(56, 56, 128),
                   pl.BlockSpec((1, 1), lambda i: (0, 0))],
        out_shape=[jax.ShapeDtypeStruct((B, 56, 56, 128), f32),
                   jax.ShapeDtypeStruct((1, 1), f32)],
        scratch_shapes=[pltpu.VMEM((58, 58, 128), f32),
                        pltpu.VMEM((56, 56, 1152), f32)],
    )(h2, wm3, b3, r1a, r1b, r2a, r2b, pre, preb, codebook, cbt,
      d1m, db1, dr1a, dr1b, dr2a, dr2b)

    loss = loss_raw[0, 0] * (1.0 + _BETA) / (B * 3136 * _EMB_DIM)

    midp = jnp.pad(mid, ((0, 0), (1, 1), (1, 1), (0, 0)))

    t1 = pl.pallas_call(
        _tconv_body(64, True),
        grid=(B,),
        in_specs=[_img_spec(58, 58, 128), _fix((4, 4, 128, 64)),
                  _fix((1, 64))],
        out_specs=_img_spec(56, 56, 256),
        out_shape=jax.ShapeDtypeStruct((B, 56, 56, 256), f32),
    )(midp, wt1, tb1)

    u1 = _d2s(t1, 64)
    u1p = jnp.pad(u1, ((0, 0), (1, 1), (1, 1), (0, 0)))

    t2 = pl.pallas_call(
        _tconv4x4_body,
        grid=(B,),
        in_specs=[_img_spec(114, 114, 64), _fix((9, 64, 12)),
                  _fix((1, 12))],
        out_specs=_img_spec(112, 112, 12),
        out_shape=jax.ShapeDtypeStruct((B, 112, 112, 12), f32),
    )(u1p, w5, tb2)

    x_recon = _d2s(t2, 3).transpose(0, 3, 1, 2)
    return loss, x_recon


# speedup vs baseline: 9.7676x; 9.7676x over previous
import jax
import jax.numpy as jnp
from jax.experimental import pallas as pl
from jax.experimental.pallas import tpu as pltpu

_NUM_EMB = 512
_EMB_DIM = 64
_BETA = 0.25


def _k1_body(x_ref, w_ref, b_ref, o_ref):
    acc = jnp.zeros((112 * 112, 64), jnp.float32)
    for a in range(2):
        for b in range(2):
            xs = x_ref[0, a:a + 112, b:b + 112, :].reshape(112 * 112, 12)
            acc += jnp.dot(xs, w_ref[a * 2 + b],
                           preferred_element_type=jnp.float32)
    o_ref[0] = jnp.maximum(acc + b_ref[0], 0.0).reshape(112, 112, 64)


def _k2_body(x_ref, w_ref, b_ref, o_ref, sc):
    for ky in range(4):
        for kx in range(4):
            a, py = ky // 2, ky % 2
            b, px = kx // 2, kx % 2
            c0 = (py * 2 + px) * 64
            k0 = (ky * 4 + kx) * 64
            sc[:, :, k0:k0 + 64] = x_ref[0, a:a + 56, b:b + 56, c0:c0 + 64]
    acc = jnp.dot(sc[...].reshape(3136, 1024), w_ref[...],
                  preferred_element_type=jnp.float32)
    o_ref[0] = jnp.maximum(acc + b_ref[0], 0.0).reshape(56, 56, 128)


def _mid_body(h0_ref, wm3_ref, b3_ref, r1a_ref, r1b_ref, r2a_ref, r2b_ref,
              pre_ref, preb_ref, cb_ref, cbt_ref, d1_ref, db1_ref,
              dr1a_ref, dr1b_ref, dr2a_ref, dr2b_ref,
              out_ref, loss_ref, sp, sc):
    i = pl.program_id(0)

    @pl.when(i == 0)
    def _init():
        loss_ref[...] = jnp.zeros((1, 1), jnp.float32)

    sp[...] = jnp.zeros((58, 58, 128), jnp.float32)

    def conv3(w_ref, cin, cout):
        for dy in range(3):
            for dx in range(3):
                k0 = (dy * 3 + dx) * cin
                sc[:, :, k0:k0 + cin] = sp[dy:dy + 56, dx:dx + 56, 0:cin]
        patches = sc[:, :, 0:9 * cin].reshape(3136, 9 * cin)
        return jnp.dot(patches, w_ref[...], preferred_element_type=jnp.float32)

    def res_block(h, ra_ref, rb_ref):
        t = jnp.maximum(h, 0.0)
        sp[1:57, 1:57, :] = t.reshape(56, 56, 128)
        u = jnp.maximum(conv3(ra_ref, 128, 32), 0.0)
        return h + jnp.dot(u, rb_ref[...], preferred_element_type=jnp.float32)

    sp[1:57, 1:57, :] = h0_ref[0]
    h = conv3(wm3_ref, 128, 128) + b3_ref[0]
    h = res_block(h, r1a_ref, r1b_ref)
    h = res_block(h, r2a_ref, r2b_ref)
    h = jnp.maximum(h, 0.0)

    z = jnp.dot(h, pre_ref[...], preferred_element_type=jnp.float32) + preb_ref[0]

    cb = cb_ref[...]
    cb2 = jnp.sum(cb * cb, axis=1)
    sse = jnp.zeros((1, 1), jnp.float32)
    for c in range(4):
        zc = z[c * 784:(c + 1) * 784, :]
        z2 = jnp.sum(zc * zc, axis=1, keepdims=True)
        d = (z2 + cb2[None, :]) - 2.0 * jnp.dot(zc, cbt_ref[...],
                                                preferred_element_type=jnp.float32)
        m = jnp.min(d, axis=1, keepdims=True)
        iota = jax.lax.broadcasted_iota(jnp.int32, (784, _NUM_EMB), 1)
        masked = jnp.where(d <= m, iota, _NUM_EMB)
        idx = jnp.min(masked, axis=1, keepdims=True)
        onehot = jnp.where(iota == idx, 1.0, 0.0)
        qc = jnp.dot(onehot, cb, preferred_element_type=jnp.float32)
        diff = qc - zc
        sse += jnp.sum(diff * diff).reshape(1, 1)
        sp[1 + c * 14:1 + (c + 1) * 14, 1:57, 0:64] = qc.reshape(14, 56, 64)
    loss_ref[...] += sse

    h = conv3(d1_ref, 64, 128) + db1_ref[0]
    h = res_block(h, dr1a_ref, dr1b_ref)
    h = res_block(h, dr2a_ref, dr2b_ref)
    h = jnp.maximum(h, 0.0)
    out_ref[0] = h.reshape(56, 56, 128)


def _tconv_body(cout, relu):
    def body(x_ref, w_ref, b_ref, o_ref):
        _, H, _, _ = o_ref.shape
        cin = x_ref.shape[3]
        for r in range(2):
            for s in range(2):
                acc = jnp.zeros((H * H, cout), jnp.float32)
                for a in range(2):
                    for b in range(2):
                        xs = x_ref[0, r + a:r + a + H, s + b:s + b + H, :]
                        acc += jnp.dot(xs.reshape(H * H, cin),
                                       w_ref[r * 2 + s, a * 2 + b],
                                       preferred_element_type=jnp.float32)
                acc = acc + b_ref[0]
                if relu:
                    acc = jnp.maximum(acc, 0.0)
                o_ref[0, :, :, (r * 2 + s) * cout:(r * 2 + s + 1) * cout] = (
                    acc.reshape(H, H, cout))
    return body


def _tconv4x4_body(x_ref, w_ref, b_ref, o_ref):
    cin = x_ref.shape[3]
    _, H, _, cq = o_ref.shape
    acc = jnp.zeros((H * H, cq), jnp.float32)
    for dy in range(3):
        for dx in range(3):
            xs = x_ref[0, dy:dy + H, dx:dx + H, :]
            acc += jnp.dot(xs.reshape(H * H, cin), w_ref[dy * 3 + dx],
                           preferred_element_type=jnp.float32)
    o_ref[0] = (acc + b_ref[0]).reshape(H, H, cq)


def _s2d(t):
    b, h, w, c = t.shape
    t = t.reshape(b, h // 2, 2, w // 2, 2, c)
    t = t.transpose(0, 1, 3, 2, 4, 5)
    return t.reshape(b, h // 2, w // 2, 4 * c)


def _d2s(t, c):
    b, h, w, _ = t.shape
    t = t.reshape(b, h, w, 2, 2, c)
    t = t.transpose(0, 1, 3, 2, 4, 5)
    return t.reshape(b, 2 * h, 2 * w, c)


def _s2d_w(w):
    o, i = w.shape[0], w.shape[1]
    t = w.transpose(2, 3, 1, 0)
    t = t.reshape(2, 2, 2, 2, i, o)
    t = t.transpose(0, 2, 1, 3, 4, 5)
    return t.reshape(4, 4 * i, o)


def _lin_w(w):
    o, i, kh, kw = w.shape
    return w.transpose(2, 3, 1, 0).reshape(kh * kw * i, o)


def _tconv_w(w):
    o, i = w.shape[0], w.shape[1]
    wf = w[:, :, ::-1, ::-1]
    t = wf.transpose(2, 3, 1, 0)
    t = t.reshape(2, 2, 2, 2, i, o)
    t = t.transpose(1, 3, 0, 2, 4, 5)
    return t.reshape(4, 4, i, o)


def _img_spec(h, w, c):
    return pl.BlockSpec((1, h, w, c), lambda i: (i, 0, 0, 0))


def _fix(shape):
    nd = len(shape)
    return pl.BlockSpec(shape, lambda i, _nd=nd: (0,) * _nd)


def kernel(x, enc_w1, enc_w2, enc_w3, enc_r1a, enc_r1b, enc_r2a, enc_r2b,
           pre_w, dec_w1, dec_r1a, dec_r1b, dec_r2a, dec_r2b,
           dect_w1, dect_w2, enc_b1, enc_b2, enc_b3, pre_b,
           dec_b1, dect_b1, dect_b2, codebook):
    f32 = jnp.float32
    B = x.shape[0]

    w1m = _s2d_w(enc_w1)
    w2m = _lin_w(enc_w2)
    wm3 = _lin_w(enc_w3)
    r1a = _lin_w(enc_r1a)
    r2a = _lin_w(enc_r2a)
    r1b = enc_r1b[:, :, 0, 0].T
    r2b = enc_r2b[:, :, 0, 0].T
    pre = pre_w[:, :, 0, 0].T
    d1m = _lin_w(dec_w1)
    dr1a = _lin_w(dec_r1a)
    dr2a = _lin_w(dec_r2a)
    dr1b = dec_r1b[:, :, 0, 0].T
    dr2b = dec_r2b[:, :, 0, 0].T
    wt1 = _tconv_w(dect_w1)
    wt2 = _tconv_w(dect_w2)
    w5 = jnp.zeros((4, 4, 64, 12), jnp.float32)
    for r in range(2):
        for s in range(2):
            for a in range(2):
                for b in range(2):
                    w5 = w5.at[r + a, s + b, :, (r * 2 + s) * 3:(r * 2 + s) * 3 + 3].set(
                        wt2[r * 2 + s, a * 2 + b])
    w5 = w5[:3, :3].reshape(9, 64, 12)
    cbt = codebook.T

    b1 = enc_b1.reshape(1, 64)
    b2 = enc_b2.reshape(1, 128)
    b3 = enc_b3.reshape(1, 128)
    preb = pre_b.reshape(1, 64)
    db1 = dec_b1.reshape(1, 128)
    tb1 = dect_b1.reshape(1, 64)
    tb2 = jnp.tile(dect_b2, 4).reshape(1, 12)

    xn = x.transpose(0, 2, 3, 1)
    xs = _s2d(jnp.pad(xn, ((0, 0), (1, 1), (1, 1), (0, 0))))

    h1 = pl.pallas_call(
        _k1_body,
        grid=(B,),
        in_specs=[_img_spec(113, 113, 12), _fix((4, 12, 64)), _fix((1, 64))],
        out_specs=_img_spec(112, 112, 64),
        out_shape=jax.ShapeDtypeStruct((B, 112, 112, 64), f32),
    )(xs, w1m, b1)

    h1s = _s2d(jnp.pad(h1, ((0, 0), (1, 1), (1, 1), (0, 0))))

    h2 = pl.pallas_call(
        _k2_body,
        grid=(B,),
        in_specs=[_img_spec(57, 57, 256), _fix((1024, 128)), _fix((1, 128))],
        out_specs=_img_spec(56, 56, 128),
        out_shape=jax.ShapeDtypeStruct((B, 56, 56, 128), f32),
        scratch_shapes=[pltpu.VMEM((56, 56, 1024), f32)],
    )(h1s, w2m, b2)

    mid, loss_raw = pl.pallas_call(
        _mid_body,
        grid=(B,),
        in_specs=[
            _img_spec(56, 56, 128),
            _fix((1152, 128)), _fix((1, 128)),
            _fix((1152, 32)), _fix((32, 128)),
            _fix((1152, 32)), _fix((32, 128)),
            _fix((128, 64)), _fix((1, 64)),
            _fix((_NUM_EMB, _EMB_DIM)), _fix((_EMB_DIM, _NUM_EMB)),
            _fix((576, 128)), _fix((1, 128)),
            _fix((1152, 32)), _fix((32, 128)),
            _fix((1152, 32)), _fix((32, 128)),
        ],
        out_specs=[_img_spec(56, 56, 128),
                   pl.BlockSpec((1, 1), lambda i: (0, 0))],
        out_shape=[jax.ShapeDtypeStruct((B, 56, 56, 128), f32),
                   jax.ShapeDtypeStruct((1, 1), f32)],
        scratch_shapes=[pltpu.VMEM((58, 58, 128), f32),
                        pltpu.VMEM((56, 56, 1152), f32)],
    )(h2, wm3, b3, r1a, r1b, r2a, r2b, pre, preb, codebook, cbt,
      d1m, db1, dr1a, dr1b, dr2a, dr2b)

    loss = loss_raw[0, 0] * (1.0 + _BETA) / (B * 3136 * _EMB_DIM)

    midp = jnp.pad(mid, ((0, 0), (1, 1), (1, 1), (0, 0)))

    t1 = pl.pallas_call(
        _tconv_body(64, True),
        grid=(B,),
        in_specs=[_img_spec(58, 58, 128), _fix((4, 4, 128, 64)),
                  _fix((1, 64))],
        out_specs=_img_spec(56, 56, 256),
        out_shape=jax.ShapeDtypeStruct((B, 56, 56, 256), f32),
    )(midp, wt1, tb1)

    u1 = _d2s(t1, 64)
    u1p = jnp.pad(u1, ((0, 0), (1, 1), (1, 1), (0, 0)))

    t2 = pl.pallas_call(
        _tconv4x4_body,
        grid=(B,),
        in_specs=[_img_spec(114, 114, 64), _fix((9, 64, 12)),
                  _fix((1, 12))],
        out_specs=_img_spec(112, 112, 12),
        out_shape=jax.ShapeDtypeStruct((B, 112, 112, 12), f32),
    )(u1p, w5, tb2)

    x_recon = _d2s(t2, 3).transpose(0, 3, 1, 2)
    return loss, x_recon
